# baseline (speedup 1.0000x reference)
import functools

import jax
import jax.numpy as jnp
from jax import lax
from jax.experimental import pallas as pl
from jax.experimental.pallas import tpu as pltpu
from jax.experimental.pallas import tpu_sc as plsc

jax.config.update("jax_enable_x64", True)

_N_TOKENS = 1024
_L = 16
_CHUNK = 16384
_UNROLL = 4


def _sc_call(scal, flat):
    info = plsc.get_sparse_core_info()
    nc, ns = info.num_cores, info.num_subcores
    nw = nc * ns
    e = flat.shape[0]
    per_w = e // nw
    n_chunks = per_w // _CHUNK

    mesh = plsc.VectorSubcoreMesh(core_axis_name="c", subcore_axis_name="s")

    @functools.partial(
        pl.kernel,
        mesh=mesh,
        out_type=jax.ShapeDtypeStruct((e,), jnp.int32),
        scratch_types=[
            pltpu.VMEM((2, _CHUNK), jnp.float32),
            pltpu.VMEM((2, _CHUNK), jnp.int32),
            pltpu.VMEM((2 * _L,), jnp.float32),
            pltpu.SemaphoreType.DMA,
            pltpu.SemaphoreType.DMA,
            pltpu.SemaphoreType.DMA,
            pltpu.SemaphoreType.DMA,
        ],
    )
    def k(scal_hbm, x_hbm, out_hbm, in_v, out_v, scal_v,
          sem_i0, sem_i1, sem_o0, sem_o1):
        wid = lax.axis_index("s") * nc + lax.axis_index("c")
        base = wid * per_w
        pltpu.sync_copy(scal_hbm, scal_v)
        min_v = scal_v[pl.ds(0, _L)]
        delta_v = scal_v[pl.ds(_L, _L)]
        sem_i = (sem_i0, sem_i1)
        sem_o = (sem_o0, sem_o1)

        def in_cp(g, b):
            return pltpu.make_async_copy(
                x_hbm.at[pl.ds(base + g * _CHUNK, _CHUNK)],
                in_v.at[b], sem_i[b])

        def out_cp(g, b):
            return pltpu.make_async_copy(
                out_v.at[b],
                out_hbm.at[pl.ds(base + g * _CHUNK, _CHUNK)], sem_o[b])

        in_cp(0, 0).start()

        def compute(b):
            def body(i, _):
                o = i * (_L * _UNROLL)
                for j in range(_UNROLL):
                    v = in_v[b, pl.ds(o + j * _L, _L)]
                    y = (v - min_v) / delta_v
                    y = jnp.minimum(jnp.maximum(y, 0.0),
                                    jnp.float32(_N_TOKENS - 1))
                    out_v[b, pl.ds(o + j * _L, _L)] = y.astype(jnp.int32)
                return 0
            lax.fori_loop(0, _CHUNK // (_L * _UNROLL), body, 0)

        def step(gg, _):
            for b in range(2):
                g = 2 * gg + b

                @pl.when(g + 1 < n_chunks)
                def _():
                    in_cp(g + 1, 1 - b).start()

                in_cp(g, b).wait()

                @pl.when(g >= 2)
                def _():
                    out_cp(g - 2, b).wait()

                compute(b)
                out_cp(g, b).start()
            return 0

        lax.fori_loop(0, n_chunks // 2, step, 0)
        out_cp(n_chunks - 2, 0).wait()
        out_cp(n_chunks - 1, 1).wait()

    return k(scal, flat)


def kernel(input, min_val, delta):
    m, n = input.shape
    with jax.enable_x64(False):
        minf = min_val.astype(jnp.float32)
        deltaf = delta.astype(jnp.float32)
        scal = jnp.concatenate([jnp.broadcast_to(minf, (_L,)),
                                jnp.broadcast_to(deltaf, (_L,))])
        out = _sc_call(scal, input.reshape(m * n))
    tok = lax.bitcast_convert_type(out.reshape(m, n), jnp.uint32)
    return lax.bitcast_convert_type(tok.astype(jnp.uint64), jnp.int64)


# speedup vs baseline: 1.0976x; 1.0976x over previous
import functools

import jax
import jax.numpy as jnp
from jax import lax
from jax.experimental import pallas as pl
from jax.experimental.pallas import tpu as pltpu
from jax.experimental.pallas import tpu_sc as plsc

jax.config.update("jax_enable_x64", True)

_N_TOKENS = 1024
_L = 16
_CHUNK = 16384
_UNROLL = 8


def _sc_call(scal, flat):
    info = plsc.get_sparse_core_info()
    nc, ns = info.num_cores, info.num_subcores
    nw = nc * ns
    e = flat.shape[0]
    per_w = e // nw
    n_chunks = per_w // _CHUNK

    mesh = plsc.VectorSubcoreMesh(core_axis_name="c", subcore_axis_name="s")

    @functools.partial(
        pl.kernel,
        mesh=mesh,
        out_type=jax.ShapeDtypeStruct((e,), jnp.int32),
        scratch_types=[
            pltpu.VMEM((2, _CHUNK), jnp.float32),
            pltpu.VMEM((2, _CHUNK), jnp.int32),
            pltpu.VMEM((2 * _L,), jnp.float32),
            pltpu.SemaphoreType.DMA,
            pltpu.SemaphoreType.DMA,
            pltpu.SemaphoreType.DMA,
            pltpu.SemaphoreType.DMA,
        ],
    )
    def k(scal_hbm, x_hbm, out_hbm, in_v, out_v, scal_v,
          sem_i0, sem_i1, sem_o0, sem_o1):
        wid = lax.axis_index("s") * nc + lax.axis_index("c")
        base = wid * per_w
        pltpu.sync_copy(scal_hbm, scal_v)
        min_v = scal_v[pl.ds(0, _L)]
        dinv_v = scal_v[pl.ds(_L, _L)]
        sem_i = (sem_i0, sem_i1)
        sem_o = (sem_o0, sem_o1)

        def in_cp(g, b):
            return pltpu.make_async_copy(
                x_hbm.at[pl.ds(base + g * _CHUNK, _CHUNK)],
                in_v.at[b], sem_i[b])

        def out_cp(g, b):
            return pltpu.make_async_copy(
                out_v.at[b],
                out_hbm.at[pl.ds(base + g * _CHUNK, _CHUNK)], sem_o[b])

        in_cp(0, 0).start()

        def compute(b):
            def body(i, _):
                o = i * (_L * _UNROLL)
                for j in range(_UNROLL):
                    v = in_v[b, pl.ds(o + j * _L, _L)]
                    y = (v - min_v) * dinv_v
                    y = jnp.minimum(jnp.maximum(y, 0.0),
                                    jnp.float32(_N_TOKENS - 1))
                    out_v[b, pl.ds(o + j * _L, _L)] = y.astype(jnp.int32)
                return 0
            lax.fori_loop(0, _CHUNK // (_L * _UNROLL), body, 0)

        def step(gg, _):
            for b in range(2):
                g = 2 * gg + b

                @pl.when(g + 1 < n_chunks)
                def _():
                    in_cp(g + 1, 1 - b).start()

                in_cp(g, b).wait()

                @pl.when(g >= 2)
                def _():
                    out_cp(g - 2, b).wait()

                compute(b)
                out_cp(g, b).start()
            return 0

        lax.fori_loop(0, n_chunks // 2, step, 0)
        out_cp(n_chunks - 2, 0).wait()
        out_cp(n_chunks - 1, 1).wait()

    return k(scal, flat)


def kernel(input, min_val, delta):
    m, n = input.shape
    with jax.enable_x64(False):
        minf = min_val.astype(jnp.float32)
        dinvf = (jnp.float32(1.0) / delta).astype(jnp.float32)
        scal = jnp.concatenate([jnp.broadcast_to(minf, (_L,)),
                                jnp.broadcast_to(dinvf, (_L,))])
        out = _sc_call(scal, input.reshape(m * n))
    tok = lax.bitcast_convert_type(out.reshape(m, n), jnp.uint32)
    return lax.bitcast_convert_type(tok.astype(jnp.uint64), jnp.int64)


# speedup vs baseline: 1.3078x; 1.1915x over previous
import jax
import jax.numpy as jnp
from jax import lax
from jax.experimental import pallas as pl
from jax.experimental.pallas import tpu as pltpu

jax.config.update("jax_enable_x64", True)

_N_TOKENS = 1024


def _body(scal_ref, x_ref, out_ref):
    min_val = scal_ref[0, 0]
    delta = scal_ref[0, 1]
    y = (x_ref[...] - min_val) / delta
    y = jnp.minimum(jnp.maximum(y, 0.0), jnp.float32(_N_TOKENS - 1))
    out_ref[...] = y.astype(jnp.uint32)


def kernel(input, min_val, delta):
    m, n = input.shape
    bm = 256
    grid = (m // bm,)
    scal = jnp.stack([min_val.astype(jnp.float32),
                      delta.astype(jnp.float32)]).reshape(1, 2)
    out = pl.pallas_call(
        _body,
        grid=grid,
        in_specs=[
            pl.BlockSpec((1, 2), lambda i: (jnp.int32(0), jnp.int32(0)),
                         memory_space=pltpu.SMEM),
            pl.BlockSpec((bm, n), lambda i: (jnp.int32(i), jnp.int32(0))),
        ],
        out_specs=pl.BlockSpec((bm, n),
                               lambda i: (jnp.int32(i), jnp.int32(0))),
        out_shape=jax.ShapeDtypeStruct((m, n), jnp.uint32),
    )(scal, input)
    return lax.bitcast_convert_type(out.astype(jnp.uint64), jnp.int64)
